# baseline (speedup 1.0000x reference)
import functools

import jax
import jax.numpy as jnp
from jax import lax
from jax.experimental import pallas as pl
from jax.experimental.pallas import tpu as pltpu
from jax.experimental.pallas import tpu_sc as plsc

NC, NS = 2, 16
NW = NC * NS


def _sc_user_item(user_id, target_item, user_table_flat, item_table_flat,
                  B, D):
  ROWS = B // NW
  KB = 16

  mesh = plsc.VectorSubcoreMesh(core_axis_name="c", subcore_axis_name="s",
                                num_cores=NC, num_subcores=NS)

  @functools.partial(
      pl.kernel,
      out_type=(jax.ShapeDtypeStruct((B, D), jnp.float32),
                jax.ShapeDtypeStruct((B, D), jnp.float32)),
      mesh=mesh,
      compiler_params=pltpu.CompilerParams(use_tc_tiling_on_sc=False),
      scratch_types=[
          pltpu.VMEM((ROWS + 16,), jnp.int32),
          pltpu.VMEM((ROWS, D), jnp.float32),
          pltpu.SemaphoreType.DMA,
      ],
  )
  def k(uid_hbm, tid_hbm, utab_hbm, itab_hbm,
        uemb_out, iemb_out, sidx_v, rows_v, sem0):
    c = lax.axis_index("c")
    s = lax.axis_index("s")
    wid = s * NC + c
    base = wid * ROWS

    for id_hbm, tab_hbm, out_hbm in ((uid_hbm, utab_hbm, uemb_out),
                                     (tid_hbm, itab_hbm, iemb_out)):
      pltpu.sync_copy(id_hbm.at[pl.ds(base, ROWS)], sidx_v.at[pl.ds(0, ROWS)])

      def io_body(t, carry):
        r0 = t * KB
        idv = sidx_v[pl.ds(r0, KB)]
        cps = []
        for u in range(KB):
          cps.append(pltpu.async_copy(tab_hbm.at[pl.ds(idv[u] * D, D)],
                                      rows_v.at[r0 + u, :], sem0))
        for cp in cps:
          cp.wait()
        return carry

      lax.fori_loop(0, ROWS // KB, io_body, 0)
      pltpu.sync_copy(rows_v, out_hbm.at[pl.ds(base, ROWS)])

  return k(user_id, target_item, user_table_flat, item_table_flat)


def _sc_hist_pool(hist_idx, seq_len, hist_table, B, L, D):
  ROWS = B // NW
  NB = 16
  NBLK = ROWS // NB
  SEG = 40
  NSEG = L // SEG

  mesh = plsc.VectorSubcoreMesh(core_axis_name="c", subcore_axis_name="s",
                                num_cores=NC, num_subcores=NS)

  @functools.partial(
      pl.kernel,
      out_type=jax.ShapeDtypeStruct((B, D), jnp.float32),
      mesh=mesh,
      compiler_params=pltpu.CompilerParams(use_tc_tiling_on_sc=False),
      scratch_types=[
          pltpu.VMEM((2, NB, L), jnp.int32),
          pltpu.VMEM((2, NB * L, D), jnp.float32),
          pltpu.VMEM((ROWS, D), jnp.float32),
          pltpu.VMEM((ROWS + 16,), jnp.int32),
          pltpu.SemaphoreType.DMA,
          pltpu.SemaphoreType.DMA,
      ],
  )
  def k(hist_hbm, slen_hbm, htab_hbm, sum_out,
        idx_v, rows_v, pool_v, sidx_v, sem0, sem1):
    c = lax.axis_index("c")
    s = lax.axis_index("s")
    wid = s * NC + c
    base = wid * ROWS

    pltpu.sync_copy(slen_hbm.at[pl.ds(base, ROWS)], sidx_v.at[pl.ds(0, ROWS)])

    def copy_idx(blk, slot):
      row0 = base + blk * NB
      pltpu.sync_copy(hist_hbm.at[pl.ds(row0, NB), :], idx_v.at[slot])

    def chunk_cps(blk, slot, sem):
      slv = sidx_v[pl.ds(blk * NB, NB)]
      out = []
      for r in range(NB):
        sl = slv[r]
        for cseg in range(NSEG):
          cp = pltpu.make_async_copy(
              htab_hbm.at[idx_v.at[slot, r, pl.ds(cseg * SEG, SEG)]],
              rows_v.at[slot, pl.ds(r * L + cseg * SEG, SEG), :],
              sem)
          out.append((sl > cseg * SEG, cp))
      return out

    def fire(blk, slot, sem):
      for cond, cp in chunk_cps(blk, slot, sem):
        @pl.when(cond)
        def _():
          cp.start()

    def drain(blk, slot, sem):
      for cond, cp in chunk_cps(blk, slot, sem):
        @pl.when(cond)
        def _():
          cp.wait()

    def compute(blk, slot):
      def row_body(r, carry):
        grow = blk * NB + r
        sl = sidx_v[pl.ds(grow, 16)][0]
        off = r * L

        for cseg in range(NSEG):
          @pl.when(sl > cseg * SEG)
          def _():
            a = [jnp.zeros((D,), jnp.float32) for _ in range(4)]
            for j in range(cseg * SEG, (cseg + 1) * SEG):
              row = rows_v[slot, off + j, :]
              m = (j < sl).astype(jnp.float32)
              a[j % 4] = a[j % 4] + row * m
            pool_v[grow, :] = ((a[0] + a[1]) + (a[2] + a[3])
                               + pool_v[grow, :])
        return carry

      lax.fori_loop(0, NB, row_body, 0)

    def zero_body(t, carry):
      pool_v[pl.ds(t * 16, 16), :] = jnp.zeros((16, D), jnp.float32)
      return carry
    lax.fori_loop(0, ROWS // 16, zero_body, 0)

    copy_idx(0, 0)
    fire(0, 0, sem0)

    def blk_body(b2, carry):
      blk0 = 2 * b2
      blk1 = blk0 + 1
      copy_idx(blk1, 1)
      fire(blk1, 1, sem1)
      drain(blk0, 0, sem0)
      compute(blk0, 0)

      @pl.when(blk1 + 1 < NBLK)
      def _():
        copy_idx(blk1 + 1, 0)
        fire(blk1 + 1, 0, sem0)

      drain(blk1, 1, sem1)
      compute(blk1, 1)
      return carry

    lax.fori_loop(0, NBLK // 2, blk_body, 0)
    pltpu.sync_copy(pool_v, sum_out.at[pl.ds(base, ROWS)])

  return k(hist_idx, seq_len, hist_table)


def _tc_mlp(user_emb, pooled_sum, item_emb, slenf, W1, b1, W2, b2,
            B, D, H1, H2):
  BT = 2048

  def body(ue, ps, ie, sl, w1, b1r, w2, b2r, out):
    denom = jnp.maximum(sl[...], 1.0)
    x = jnp.concatenate([ue[...], ps[...] / denom], axis=1)
    h = jnp.maximum(jnp.dot(x, w1[...].T) + b1r[...], 0.0)
    u = jnp.maximum(jnp.dot(h, w2[...].T) + b2r[...], 0.0)
    out[...] = jnp.sum(u * ie[...], axis=1, keepdims=True)

  return pl.pallas_call(
      body,
      grid=(B // BT,),
      in_specs=[
          pl.BlockSpec((BT, D), lambda i: (i, 0)),
          pl.BlockSpec((BT, D), lambda i: (i, 0)),
          pl.BlockSpec((BT, D), lambda i: (i, 0)),
          pl.BlockSpec((BT, 1), lambda i: (i, 0)),
          pl.BlockSpec((H1, 2 * D), lambda i: (0, 0)),
          pl.BlockSpec((1, H1), lambda i: (0, 0)),
          pl.BlockSpec((H2, H1), lambda i: (0, 0)),
          pl.BlockSpec((1, H2), lambda i: (0, 0)),
      ],
      out_specs=pl.BlockSpec((BT, 1), lambda i: (i, 0)),
      out_shape=jax.ShapeDtypeStruct((B, 1), jnp.float32),
  )(user_emb, pooled_sum, item_emb, slenf, W1, b1, W2, b2)


def kernel(user_id, hist_item_seq, target_item, seq_len,
           user_table, hist_table, item_table, W1, b1, W2, b2):
  B, L = hist_item_seq.shape
  V, D = hist_table.shape
  H1 = W1.shape[0]
  H2 = W2.shape[0]

  hist_idx = hist_item_seq.astype(jnp.int32)
  uid = user_id.astype(jnp.int32)
  tid = target_item.astype(jnp.int32)
  sli = seq_len.astype(jnp.int32)

  uemb, iemb = _sc_user_item(uid, tid, user_table.reshape(V * D),
                             item_table.reshape(V * D), B, D)
  pooled_sum = _sc_hist_pool(hist_idx, sli, hist_table, B, L, D)

  slenf = seq_len.astype(jnp.float32).reshape(B, 1)
  score = _tc_mlp(uemb, pooled_sum, iemb, slenf,
                  W1, b1.reshape(1, H1), W2, b2.reshape(1, H2),
                  B, D, H1, H2)
  return score.reshape(B)


# speedup vs baseline: 1.2371x; 1.2371x over previous
import functools

import jax
import jax.numpy as jnp
from jax import lax
from jax.experimental import pallas as pl
from jax.experimental.pallas import tpu as pltpu
from jax.experimental.pallas import tpu_sc as plsc

NC, NS = 2, 16
NW = NC * NS


def _sc_user_item(user_id, target_item, user_table, item_table, B, D):
  ROWS = B // NW
  KB = 16

  mesh = plsc.VectorSubcoreMesh(core_axis_name="c", subcore_axis_name="s",
                                num_cores=NC, num_subcores=NS)

  @functools.partial(
      pl.kernel,
      out_type=(jax.ShapeDtypeStruct((B, D), jnp.float32),
                jax.ShapeDtypeStruct((B, D), jnp.float32)),
      mesh=mesh,
      compiler_params=pltpu.CompilerParams(use_tc_tiling_on_sc=True),
      scratch_types=[
          pltpu.VMEM((ROWS + 16,), jnp.int32),
          pltpu.VMEM((ROWS, D), jnp.float32),
          pltpu.SemaphoreType.DMA,
      ],
  )
  def k(uid_hbm, tid_hbm, utab_hbm, itab_hbm,
        uemb_out, iemb_out, sidx_v, rows_v, sem0):
    c = lax.axis_index("c")
    s = lax.axis_index("s")
    wid = s * NC + c
    base = wid * ROWS

    for id_hbm, tab_hbm, out_hbm in ((uid_hbm, utab_hbm, uemb_out),
                                     (tid_hbm, itab_hbm, iemb_out)):
      pltpu.sync_copy(id_hbm.at[pl.ds(base, ROWS)], sidx_v.at[pl.ds(0, ROWS)])

      def io_body(t, carry):
        r0 = t * KB
        idv = sidx_v[pl.ds(r0, KB)]
        cps = []
        for u in range(KB):
          cps.append(pltpu.async_copy(tab_hbm.at[idv[u], :],
                                      rows_v.at[r0 + u, :], sem0))
        for cp in cps:
          cp.wait()
        return carry

      lax.fori_loop(0, ROWS // KB, io_body, 0)
      pltpu.sync_copy(rows_v, out_hbm.at[pl.ds(base, ROWS)])

  return k(user_id, target_item, user_table, item_table)


def _sc_hist_pool(hist_idx, seq_len, hist_table, B, L, D):
  ROWS = B // NW
  NB = 16
  NBLK = ROWS // NB
  SEG = 40
  NSEG = L // SEG

  mesh = plsc.VectorSubcoreMesh(core_axis_name="c", subcore_axis_name="s",
                                num_cores=NC, num_subcores=NS)

  @functools.partial(
      pl.kernel,
      out_type=jax.ShapeDtypeStruct((B, D), jnp.float32),
      mesh=mesh,
      compiler_params=pltpu.CompilerParams(use_tc_tiling_on_sc=False),
      scratch_types=[
          pltpu.VMEM((2, NB, L), jnp.int32),
          pltpu.VMEM((2, NB * L, D), jnp.float32),
          pltpu.VMEM((ROWS, D), jnp.float32),
          pltpu.VMEM((ROWS + 16,), jnp.int32),
          pltpu.SemaphoreType.DMA,
          pltpu.SemaphoreType.DMA,
      ],
  )
  def k(hist_hbm, slen_hbm, htab_hbm, sum_out,
        idx_v, rows_v, pool_v, sidx_v, sem0, sem1):
    c = lax.axis_index("c")
    s = lax.axis_index("s")
    wid = s * NC + c
    base = wid * ROWS

    pltpu.sync_copy(slen_hbm.at[pl.ds(base, ROWS)], sidx_v.at[pl.ds(0, ROWS)])

    def copy_idx(blk, slot):
      row0 = base + blk * NB
      pltpu.sync_copy(hist_hbm.at[pl.ds(row0, NB), :], idx_v.at[slot])

    def chunk_cps(blk, slot, sem):
      slv = sidx_v[pl.ds(blk * NB, NB)]
      out = []
      for r in range(NB):
        sl = slv[r]
        for cseg in range(NSEG):
          cp = pltpu.make_async_copy(
              htab_hbm.at[idx_v.at[slot, r, pl.ds(cseg * SEG, SEG)]],
              rows_v.at[slot, pl.ds(r * L + cseg * SEG, SEG), :],
              sem)
          out.append((sl > cseg * SEG, cp))
      return out

    def fire(blk, slot, sem):
      for cond, cp in chunk_cps(blk, slot, sem):
        @pl.when(cond)
        def _():
          cp.start()

    def drain(blk, slot, sem):
      for cond, cp in chunk_cps(blk, slot, sem):
        @pl.when(cond)
        def _():
          cp.wait()

    def compute(blk, slot):
      def row_body(r, carry):
        grow = blk * NB + r
        sl = sidx_v[pl.ds(grow, 16)][0]
        off = r * L

        for cseg in range(NSEG):
          @pl.when(sl > cseg * SEG)
          def _():
            a = [jnp.zeros((D,), jnp.float32) for _ in range(4)]
            for j in range(cseg * SEG, (cseg + 1) * SEG):
              row = rows_v[slot, off + j, :]
              m = (j < sl).astype(jnp.float32)
              a[j % 4] = a[j % 4] + row * m
            pool_v[grow, :] = ((a[0] + a[1]) + (a[2] + a[3])
                               + pool_v[grow, :])
        return carry

      lax.fori_loop(0, NB, row_body, 0)

    def zero_body(t, carry):
      pool_v[pl.ds(t * 16, 16), :] = jnp.zeros((16, D), jnp.float32)
      return carry
    lax.fori_loop(0, ROWS // 16, zero_body, 0)

    copy_idx(0, 0)
    fire(0, 0, sem0)

    def blk_body(b2, carry):
      blk0 = 2 * b2
      blk1 = blk0 + 1
      copy_idx(blk1, 1)
      fire(blk1, 1, sem1)
      drain(blk0, 0, sem0)
      compute(blk0, 0)

      @pl.when(blk1 + 1 < NBLK)
      def _():
        copy_idx(blk1 + 1, 0)
        fire(blk1 + 1, 0, sem0)

      drain(blk1, 1, sem1)
      compute(blk1, 1)
      return carry

    lax.fori_loop(0, NBLK // 2, blk_body, 0)
    pltpu.sync_copy(pool_v, sum_out.at[pl.ds(base, ROWS)])

  return k(hist_idx, seq_len, hist_table)


def _tc_mlp(user_emb, pooled_sum, item_emb, slenf, W1, b1, W2, b2,
            B, D, H1, H2):
  BT = 2048

  def body(ue, ps, ie, sl, w1, b1r, w2, b2r, out):
    denom = jnp.maximum(sl[...], 1.0)
    x = jnp.concatenate([ue[...], ps[...] / denom], axis=1)
    h = jnp.maximum(jnp.dot(x, w1[...].T) + b1r[...], 0.0)
    u = jnp.maximum(jnp.dot(h, w2[...].T) + b2r[...], 0.0)
    out[...] = jnp.sum(u * ie[...], axis=1, keepdims=True)

  return pl.pallas_call(
      body,
      grid=(B // BT,),
      in_specs=[
          pl.BlockSpec((BT, D), lambda i: (i, 0)),
          pl.BlockSpec((BT, D), lambda i: (i, 0)),
          pl.BlockSpec((BT, D), lambda i: (i, 0)),
          pl.BlockSpec((BT, 1), lambda i: (i, 0)),
          pl.BlockSpec((H1, 2 * D), lambda i: (0, 0)),
          pl.BlockSpec((1, H1), lambda i: (0, 0)),
          pl.BlockSpec((H2, H1), lambda i: (0, 0)),
          pl.BlockSpec((1, H2), lambda i: (0, 0)),
      ],
      out_specs=pl.BlockSpec((BT, 1), lambda i: (i, 0)),
      out_shape=jax.ShapeDtypeStruct((B, 1), jnp.float32),
  )(user_emb, pooled_sum, item_emb, slenf, W1, b1, W2, b2)


def kernel(user_id, hist_item_seq, target_item, seq_len,
           user_table, hist_table, item_table, W1, b1, W2, b2):
  B, L = hist_item_seq.shape
  V, D = hist_table.shape
  H1 = W1.shape[0]
  H2 = W2.shape[0]

  hist_idx = hist_item_seq.astype(jnp.int32)
  uid = user_id.astype(jnp.int32)
  tid = target_item.astype(jnp.int32)
  sli = seq_len.astype(jnp.int32)

  uemb, iemb = _sc_user_item(uid, tid, user_table, item_table, B, D)
  pooled_sum = _sc_hist_pool(hist_idx, sli, hist_table, B, L, D)

  slenf = seq_len.astype(jnp.float32).reshape(B, 1)
  score = _tc_mlp(uemb, pooled_sum, iemb, slenf,
                  W1, b1.reshape(1, H1), W2, b2.reshape(1, H2),
                  B, D, H1, H2)
  return score.reshape(B)
